# Initial kernel scaffold; baseline (speedup 1.0000x reference)
import jax
import jax.numpy as jnp
from jax.experimental import pallas as pl


def kernel(indices, table, bias_table):
    raise NotImplementedError("write your pallas kernel here")


# speedup vs baseline: 7.1349x; 7.1349x over previous
import functools

import jax
import jax.numpy as jnp
from jax import lax
from jax.experimental import pallas as pl
from jax.experimental.pallas import tpu as pltpu
from jax.experimental.pallas import tpu_sc as plsc

_NUM_WORKERS = 32
_CHUNK = 256


@functools.lru_cache(maxsize=None)
def _build(n_rows, dim):
    nw = _NUM_WORKERS
    rows_per_w = n_rows // nw
    c = _CHUNK
    nchunks = rows_per_w // c
    assert rows_per_w % c == 0 and n_rows % nw == 0
    lanes = 16

    mesh = plsc.VectorSubcoreMesh(core_axis_name="c", subcore_axis_name="s")

    @functools.partial(
        pl.kernel,
        out_type=jax.ShapeDtypeStruct((n_rows, dim), jnp.float32),
        mesh=mesh,
        scratch_types=[
            pltpu.VMEM((c,), jnp.int32),
            pltpu.VMEM((c, dim), jnp.float32),
            pltpu.VMEM((c,), jnp.float32),
            pltpu.SemaphoreType.DMA,
            pltpu.SemaphoreType.DMA,
        ],
    )
    def sc_gather(idx_hbm, table_hbm, bias_hbm, out_hbm,
                  idx_v, rows_v, bias_v, sem_r, sem_b):
        wid = lax.axis_index("s") * 2 + lax.axis_index("c")
        base = wid * rows_per_w

        def chunk_body(g, carry):
            off = base + g * c
            pltpu.sync_copy(idx_hbm.at[pl.ds(off, c)], idx_v)
            row_dma = pltpu.make_async_copy(table_hbm.at[idx_v], rows_v, sem_r)
            bias_dma = pltpu.make_async_copy(bias_hbm.at[idx_v], bias_v, sem_b)
            row_dma.start()
            bias_dma.start()
            row_dma.wait()
            bias_dma.wait()

            def grp_body(t, carry2):
                bvec = bias_v[pl.ds(t * lanes, lanes)]
                for k in range(lanes):
                    b = bvec[k]
                    r = t * lanes + k
                    for j in range(dim // lanes):
                        sl = pl.ds(j * lanes, lanes)
                        rows_v[r, sl] = rows_v[r, sl] + b
                return carry2

            lax.fori_loop(0, c // lanes, grp_body, 0)
            pltpu.sync_copy(rows_v, out_hbm.at[pl.ds(off, c)])
            return carry

        lax.fori_loop(0, nchunks, chunk_body, 0)

    return sc_gather


def kernel(indices, table, bias_table):
    b, l = indices.shape
    _, dim = table.shape
    flat_idx = indices.reshape(b * l)
    flat_bias = bias_table.reshape(-1)
    out = _build(b * l, dim)(flat_idx, table, flat_bias)
    return out.reshape(b, l, dim)


# speedup vs baseline: 7.6532x; 1.0726x over previous
import functools

import jax
import jax.numpy as jnp
from jax import lax
from jax.experimental import pallas as pl
from jax.experimental.pallas import tpu as pltpu
from jax.experimental.pallas import tpu_sc as plsc

_NUM_WORKERS = 32
_CHUNK = 400


@functools.lru_cache(maxsize=None)
def _build(n_rows, dim):
    nw = _NUM_WORKERS
    rows_per_w = n_rows // nw
    c = _CHUNK
    nchunks = rows_per_w // c
    npairs = nchunks // 2
    assert rows_per_w % c == 0 and n_rows % nw == 0 and nchunks % 2 == 0
    lanes = 16

    mesh = plsc.VectorSubcoreMesh(core_axis_name="c", subcore_axis_name="s")

    @functools.partial(
        pl.kernel,
        out_type=jax.ShapeDtypeStruct((n_rows, dim), jnp.float32),
        mesh=mesh,
        scratch_types=[
            pltpu.VMEM((c,), jnp.int32),
            pltpu.VMEM((c,), jnp.int32),
            pltpu.VMEM((c, dim), jnp.float32),
            pltpu.VMEM((c, dim), jnp.float32),
            pltpu.VMEM((c,), jnp.float32),
            pltpu.VMEM((c,), jnp.float32),
            pltpu.SemaphoreType.DMA,
            pltpu.SemaphoreType.DMA,
            pltpu.SemaphoreType.DMA,
            pltpu.SemaphoreType.DMA,
            pltpu.SemaphoreType.DMA,
            pltpu.SemaphoreType.DMA,
        ],
    )
    def sc_gather(idx_hbm, table_hbm, bias_hbm, out_hbm,
                  idx0, idx1, rows0, rows1, bias0, bias1,
                  sr0, sr1, sb0, sb1, so0, so1):
        wid = lax.axis_index("s") * 2 + lax.axis_index("c")
        base = wid * rows_per_w
        slots = [(idx0, rows0, bias0, sr0, sb0, so0),
                 (idx1, rows1, bias1, sr1, sb1, so1)]

        def start_gather(g, s):
            idx_v, rows_v, bias_v, sr, sb, _ = slots[s]
            off = base + g * c
            pltpu.sync_copy(idx_hbm.at[pl.ds(off, c)], idx_v)
            pltpu.make_async_copy(table_hbm.at[idx_v], rows_v, sr).start()
            pltpu.make_async_copy(bias_hbm.at[idx_v], bias_v, sb).start()

        def wait_gather(s):
            idx_v, rows_v, bias_v, sr, sb, _ = slots[s]
            pltpu.make_async_copy(table_hbm.at[idx_v], rows_v, sr).wait()
            pltpu.make_async_copy(bias_hbm.at[idx_v], bias_v, sb).wait()

        def compute(s):
            _, rows_v, bias_v, _, _, _ = slots[s]

            def grp_body(t, carry):
                bvec = bias_v[pl.ds(t * lanes, lanes)]
                for k in range(lanes):
                    b = bvec[k]
                    r = t * lanes + k
                    for j in range(dim // lanes):
                        sl = pl.ds(j * lanes, lanes)
                        rows_v[r, sl] = rows_v[r, sl] + b
                return carry

            lax.fori_loop(0, c // lanes, grp_body, 0)

        def start_scatter(g, s):
            _, rows_v, _, _, _, so = slots[s]
            off = base + g * c
            pltpu.make_async_copy(rows_v, out_hbm.at[pl.ds(off, c)], so).start()

        def wait_scatter(s):
            _, rows_v, _, _, _, so = slots[s]
            pltpu.make_async_copy(rows_v, out_hbm.at[pl.ds(base, c)], so).wait()

        start_gather(0, 0)

        def pair_body(p, carry):
            g = p * 2
            wait_gather(0)
            compute(0)

            @pl.when(p > 0)
            def _():
                wait_scatter(1)

            start_gather(g + 1, 1)
            start_scatter(g, 0)
            wait_gather(1)
            compute(1)
            wait_scatter(0)

            @pl.when(p < npairs - 1)
            def _():
                start_gather(g + 2, 0)

            start_scatter(g + 1, 1)
            return carry

        lax.fori_loop(0, npairs, pair_body, 0)
        wait_scatter(1)

    return sc_gather


def kernel(indices, table, bias_table):
    b, l = indices.shape
    _, dim = table.shape
    flat_idx = indices.reshape(b * l)
    flat_bias = bias_table.reshape(-1)
    out = _build(b * l, dim)(flat_idx, table, flat_bias)
    return out.reshape(b, l, dim)


# speedup vs baseline: 12.4431x; 1.6259x over previous
import functools

import jax
import jax.numpy as jnp
from jax import lax
from jax.experimental import pallas as pl
from jax.experimental.pallas import tpu as pltpu
from jax.experimental.pallas import tpu_sc as plsc

_NUM_WORKERS = 32
_CHUNK = 400


@functools.lru_cache(maxsize=None)
def _build(n_seq, seq_len, dim):
    n_rows = n_seq * seq_len
    nw = _NUM_WORKERS
    rows_per_w = n_rows // nw
    c = _CHUNK
    nchunks = rows_per_w // c
    npairs = nchunks // 2
    assert rows_per_w % c == 0 and n_rows % nw == 0 and nchunks % 2 == 0
    assert c % seq_len == 0
    seq_per_chunk = c // seq_len
    lanes = 16

    mesh = plsc.VectorSubcoreMesh(core_axis_name="c", subcore_axis_name="s")

    @functools.partial(
        pl.kernel,
        out_type=jax.ShapeDtypeStruct((n_seq, seq_len, dim), jnp.float32),
        mesh=mesh,
        scratch_types=[
            pltpu.VMEM((c,), jnp.int32),
            pltpu.VMEM((c,), jnp.int32),
            pltpu.VMEM((c, dim), jnp.float32),
            pltpu.VMEM((c, dim), jnp.float32),
            pltpu.VMEM((c,), jnp.float32),
            pltpu.VMEM((c,), jnp.float32),
            pltpu.SemaphoreType.DMA,
            pltpu.SemaphoreType.DMA,
            pltpu.SemaphoreType.DMA,
            pltpu.SemaphoreType.DMA,
            pltpu.SemaphoreType.DMA,
            pltpu.SemaphoreType.DMA,
        ],
    )
    def sc_gather(idx_hbm, table_hbm, bias_hbm, out_hbm,
                  idx0, idx1, rows0, rows1, bias0, bias1,
                  sr0, sr1, sb0, sb1, so0, so1):
        wid = lax.axis_index("s") * 2 + lax.axis_index("c")
        base = wid * rows_per_w
        slots = [(idx0, rows0, bias0, sr0, sb0, so0),
                 (idx1, rows1, bias1, sr1, sb1, so1)]

        def start_gather(g, s):
            idx_v, rows_v, bias_v, sr, sb, _ = slots[s]
            off = base + g * c
            pltpu.sync_copy(idx_hbm.at[pl.ds(off, c)], idx_v)
            pltpu.make_async_copy(table_hbm.at[idx_v], rows_v, sr).start()
            pltpu.make_async_copy(bias_hbm.at[idx_v], bias_v, sb).start()

        def wait_gather(s):
            idx_v, rows_v, bias_v, sr, sb, _ = slots[s]
            pltpu.make_async_copy(table_hbm.at[idx_v], rows_v, sr).wait()
            pltpu.make_async_copy(bias_hbm.at[idx_v], bias_v, sb).wait()

        def compute(s):
            _, rows_v, bias_v, _, _, _ = slots[s]

            def grp_body(t, carry):
                bvec = bias_v[pl.ds(t * lanes, lanes)]
                for k in range(lanes):
                    b = bvec[k]
                    r = t * lanes + k
                    for j in range(dim // lanes):
                        sl = pl.ds(j * lanes, lanes)
                        rows_v[r, sl] = rows_v[r, sl] + b
                return carry

            lax.fori_loop(0, c // lanes, grp_body, 0)

        def start_scatter(g, s):
            _, rows_v, _, _, _, so = slots[s]
            seq0 = (base + g * c) // seq_len
            for q in range(seq_per_chunk):
                pltpu.make_async_copy(
                    rows_v.at[pl.ds(q * seq_len, seq_len)],
                    out_hbm.at[seq0 + q], so).start()

        def wait_scatter(s):
            _, rows_v, _, _, _, so = slots[s]
            for q in range(seq_per_chunk):
                pltpu.make_async_copy(
                    rows_v.at[pl.ds(q * seq_len, seq_len)],
                    out_hbm.at[base // seq_len + q], so).wait()

        start_gather(0, 0)

        def pair_body(p, carry):
            g = p * 2
            wait_gather(0)
            compute(0)

            @pl.when(p > 0)
            def _():
                wait_scatter(1)

            start_gather(g + 1, 1)
            start_scatter(g, 0)
            wait_gather(1)
            compute(1)
            wait_scatter(0)

            @pl.when(p < npairs - 1)
            def _():
                start_gather(g + 2, 0)

            start_scatter(g + 1, 1)
            return carry

        lax.fori_loop(0, npairs, pair_body, 0)
        wait_scatter(1)

    return sc_gather


def kernel(indices, table, bias_table):
    b, l = indices.shape
    _, dim = table.shape
    flat_idx = indices.reshape(b * l)
    flat_bias = bias_table.reshape(-1)
    return _build(b, l, dim)(flat_idx, table, flat_bias)


# speedup vs baseline: 12.4646x; 1.0017x over previous
import functools

import jax
import jax.numpy as jnp
from jax import lax
from jax.experimental import pallas as pl
from jax.experimental.pallas import tpu as pltpu
from jax.experimental.pallas import tpu_sc as plsc

_NUM_WORKERS = 32
_CHUNK = 400


@functools.lru_cache(maxsize=None)
def _build(n_seq, seq_len, dim):
    n_rows = n_seq * seq_len
    nw = _NUM_WORKERS
    rows_per_w = n_rows // nw
    c = _CHUNK
    nchunks = rows_per_w // c
    npairs = nchunks // 2
    assert rows_per_w % c == 0 and n_rows % nw == 0 and nchunks % 2 == 0
    assert c % seq_len == 0
    seq_per_chunk = c // seq_len
    lanes = 16

    mesh = plsc.VectorSubcoreMesh(core_axis_name="c", subcore_axis_name="s")

    @functools.partial(
        pl.kernel,
        out_type=jax.ShapeDtypeStruct((n_seq, seq_len, dim), jnp.float32),
        mesh=mesh,
        compiler_params=pltpu.CompilerParams(use_tc_tiling_on_sc=True),
        scratch_types=[
            pltpu.VMEM((c,), jnp.int32),
            pltpu.VMEM((c,), jnp.int32),
            pltpu.VMEM((c, dim), jnp.float32),
            pltpu.VMEM((c, dim), jnp.float32),
            pltpu.VMEM((c,), jnp.float32),
            pltpu.VMEM((c,), jnp.float32),
            pltpu.SemaphoreType.DMA,
            pltpu.SemaphoreType.DMA,
            pltpu.SemaphoreType.DMA,
            pltpu.SemaphoreType.DMA,
            pltpu.SemaphoreType.DMA,
            pltpu.SemaphoreType.DMA,
        ],
    )
    def sc_gather(idx_hbm, table_hbm, bias_hbm, out_hbm,
                  idx0, idx1, rows0, rows1, bias0, bias1,
                  sr0, sr1, sb0, sb1, so0, so1):
        wid = lax.axis_index("s") * 2 + lax.axis_index("c")
        base = wid * rows_per_w
        slots = [(idx0, rows0, bias0, sr0, sb0, so0),
                 (idx1, rows1, bias1, sr1, sb1, so1)]

        def start_gather(g, s):
            idx_v, rows_v, bias_v, sr, sb, _ = slots[s]
            off = base + g * c
            pltpu.sync_copy(idx_hbm.at[pl.ds(off, c)], idx_v)
            pltpu.make_async_copy(table_hbm.at[idx_v], rows_v, sr).start()
            pltpu.make_async_copy(bias_hbm.at[idx_v], bias_v, sb).start()

        def wait_gather(s):
            idx_v, rows_v, bias_v, sr, sb, _ = slots[s]
            pltpu.make_async_copy(table_hbm.at[idx_v], rows_v, sr).wait()
            pltpu.make_async_copy(bias_hbm.at[idx_v], bias_v, sb).wait()

        def compute(s):
            _, rows_v, bias_v, _, _, _ = slots[s]

            def grp_body(t, carry):
                bvec = bias_v[pl.ds(t * lanes, lanes)]
                for k in range(lanes):
                    b = bvec[k]
                    r = t * lanes + k
                    for j in range(dim // lanes):
                        sl = pl.ds(j * lanes, lanes)
                        rows_v[r, sl] = rows_v[r, sl] + b
                return carry

            lax.fori_loop(0, c // lanes, grp_body, 0)

        def start_scatter(g, s):
            _, rows_v, _, _, _, so = slots[s]
            seq0 = (base + g * c) // seq_len
            for q in range(seq_per_chunk):
                pltpu.make_async_copy(
                    rows_v.at[pl.ds(q * seq_len, seq_len)],
                    out_hbm.at[seq0 + q], so).start()

        def wait_scatter(s):
            _, rows_v, _, _, _, so = slots[s]
            for q in range(seq_per_chunk):
                pltpu.make_async_copy(
                    rows_v.at[pl.ds(q * seq_len, seq_len)],
                    out_hbm.at[base // seq_len + q], so).wait()

        start_gather(0, 0)

        def pair_body(p, carry):
            g = p * 2
            wait_gather(0)
            compute(0)

            @pl.when(p > 0)
            def _():
                wait_scatter(1)

            start_gather(g + 1, 1)
            start_scatter(g, 0)
            wait_gather(1)
            compute(1)
            wait_scatter(0)

            @pl.when(p < npairs - 1)
            def _():
                start_gather(g + 2, 0)

            start_scatter(g + 1, 1)
            return carry

        lax.fori_loop(0, npairs, pair_body, 0)
        wait_scatter(1)

    return sc_gather


def kernel(indices, table, bias_table):
    b, l = indices.shape
    _, dim = table.shape
    flat_idx = indices.reshape(b * l)
    flat_bias = bias_table.reshape(-1)
    return _build(b, l, dim)(flat_idx, table, flat_bias)
